# baseline (speedup 1.0000x reference)
import jax
import jax.numpy as jnp
from jax.experimental import pallas as pl


def _readout_kernel(x_ref, idx_ref, w_ref, b_ref, out_ref):
    acc = jnp.dot(x_ref[...], w_ref[...], preferred_element_type=jnp.float32)
    idx = idx_ref[...]
    T = out_ref.shape[0]
    E = out_ref.shape[2]
    for t in range(T):
        mask = idx == t
        vals = acc[:, t * E:(t + 1) * E] + b_ref[0, t]
        out_ref[t] = jnp.where(mask, vals, 0.0)


def kernel(output_latents, output_task_index, W, b):
    B, N, D = output_latents.shape
    T, _, E = W.shape
    M = B * N

    X = output_latents.reshape(M, D)
    idx = output_task_index.reshape(M, 1)
    Wf = W.transpose(1, 0, 2).reshape(D, T * E)
    bf = b.reshape(1, T, E)

    TM = 512
    grid = (M // TM,)

    out = pl.pallas_call(
        _readout_kernel,
        grid=grid,
        in_specs=[
            pl.BlockSpec((TM, D), lambda i: (i, 0)),
            pl.BlockSpec((TM, 1), lambda i: (i, 0)),
            pl.BlockSpec((D, T * E), lambda i: (0, 0)),
            pl.BlockSpec((1, T, E), lambda i: (0, 0, 0)),
        ],
        out_specs=pl.BlockSpec((T, TM, E), lambda i: (0, i, 0)),
        out_shape=jax.ShapeDtypeStruct((T, M, E), jnp.float32),
    )(X, idx, Wf, bf)
    return out.reshape(T, B, N, E)
